# baseline (speedup 1.0000x reference)
import jax
import jax.numpy as jnp
from jax.experimental import pallas as pl
from jax.experimental.pallas import tpu as pltpu


def _conv_body(x_ref, w_ref, o_ref):
    nb, g, _ = o_ref.shape
    xb = x_ref[...]
    patches = jnp.concatenate(
        [xb[:, d:d + g, :] for d in range(3)], axis=2)
    p2 = patches.reshape(nb * g, patches.shape[2])
    acc = jnp.dot(p2, w_ref[...], preferred_element_type=jnp.float32)
    o_ref[...] = acc.reshape(nb, g, o_ref.shape[2])


def _pack_weights(k1a, k1b, k2a, k2b, w):
    k = k1a.shape[0]
    ci, co = k1a.shape[1], k1a.shape[2]
    pad = (k - 1) // 2
    g2d = (jnp.einsum("tim,smo->tsio", k1a, k1b)
           + jnp.einsum("sim,tmo->tsio", k2a, k2b))

    t_ar = jnp.arange(k)
    dg = jnp.arange(3)
    hi = jnp.arange(8)
    ho = jnp.arange(8)
    tsel = (t_ar[:, None, None, None]
            == 8 * dg[None, :, None, None] + hi[None, None, :, None]
            - ho[None, None, None, :] - (8 - pad)).astype(jnp.float32)
    s_ar = jnp.arange(k)
    wi = jnp.arange(w)
    wo = jnp.arange(w)
    ssel = (s_ar[:, None, None]
            == wi[None, :, None] - wo[None, None, :] + pad).astype(jnp.float32)
    full = jnp.einsum("tdah,swv,tsio->dawihvo", tsel, ssel, g2d)
    return full.reshape(3 * 8 * w * ci, 8 * w * co)


def kernel(x_nchw, k1a, k1b, k2a, k2b):
    n, c_in, h, w = x_nchw.shape
    cc = k1a.shape[2]
    g = h // 8
    gp = g + 2
    lanes_in = 8 * w * c_in
    lanes_out = 8 * w * cc

    wmat = _pack_weights(k1a, k1b, k2a, k2b, w).astype(jnp.bfloat16)

    x = jnp.transpose(x_nchw, (0, 2, 3, 1))
    x = jnp.pad(x, ((0, 0), (8, 8), (0, 0), (0, 0)))
    x = x.reshape(n, gp, lanes_in).astype(jnp.bfloat16)

    nb = 4 if n % 4 == 0 else 1
    out = pl.pallas_call(
        _conv_body,
        out_shape=jax.ShapeDtypeStruct((n, g, lanes_out), jnp.float32),
        grid=(n // nb,),
        in_specs=[
            pl.BlockSpec((nb, gp, lanes_in), lambda i: (i, 0, 0)),
            pl.BlockSpec((3 * lanes_in, lanes_out), lambda i: (0, 0)),
        ],
        out_specs=pl.BlockSpec((nb, g, lanes_out), lambda i: (i, 0, 0)),
        compiler_params=pltpu.CompilerParams(
            dimension_semantics=("parallel",)),
    )(x, wmat)

    out = out.reshape(n, h, w, cc)
    return jnp.transpose(out, (0, 3, 1, 2))
